# Initial kernel scaffold; baseline (speedup 1.0000x reference)
import jax
import jax.numpy as jnp
from jax.experimental import pallas as pl


def kernel(x):
    raise NotImplementedError("write your pallas kernel here")


# speedup vs baseline: 3.9165x; 3.9165x over previous
import jax
import jax.numpy as jnp
from jax.experimental import pallas as pl

_SUP = 300
_EPS = 0.001
_NBINS = 2 * _SUP + 1
_R = 2048


def _twohot_block(x_ref, enc_s_ref, enc_v_ref):
    x = x_ref[0]
    enc_s = jnp.sign(x) * (jnp.sqrt(jnp.abs(x) + 1.0) - 1.0) + _EPS * x
    enc_s = jnp.clip(enc_s, -float(_SUP), float(_SUP))
    fl = jnp.floor(enc_s)
    rem = enc_s - fl
    fli = fl.astype(jnp.int32)
    idx1 = jnp.minimum(_SUP + fli + 1, 2 * _SUP)
    idx2 = _SUP + fli
    cols = jax.lax.broadcasted_iota(jnp.int32, (_R, _NBINS), 1)
    block = jnp.where(cols == idx2, 1.0 - rem,
                      jnp.where(cols == idx1, rem, 0.0))
    enc_s_ref[0] = enc_s
    enc_v_ref[0] = block


def kernel(x):
    n = x.shape[0]
    nb = n // _R
    x3 = x.reshape(nb, _R, 1)
    enc_s, enc_v = pl.pallas_call(
        _twohot_block,
        grid=(nb,),
        in_specs=[pl.BlockSpec((1, _R, 1), lambda i: (i, 0, 0))],
        out_specs=[
            pl.BlockSpec((1, _R, 1), lambda i: (i, 0, 0)),
            pl.BlockSpec((1, _R, _NBINS), lambda i: (i, 0, 0)),
        ],
        out_shape=[
            jax.ShapeDtypeStruct((nb, _R, 1), jnp.float32),
            jax.ShapeDtypeStruct((nb, _R, _NBINS), jnp.float32),
        ],
    )(x3)
    return (enc_s.reshape(n), enc_v.reshape(n, _NBINS))
